# baseline (speedup 1.0000x reference)
import functools

import jax
import jax.numpy as jnp
from jax import lax
from jax.experimental import pallas as pl
from jax.experimental.pallas import tpu as pltpu
from jax.experimental.pallas import tpu_sc as plsc

B = 16384
F = 26
V = 100000
D = 32
BF = B * F
NW = 32
PER_W = BF // NW
CH = 512
NCH = PER_W // CH
NSTREAM = CH // 128
G = (V * F) // 4


def _gather_kernel(table_hbm, idx_hbm, out_hbm, idx_v, fgrp_v, soff_v,
                   grp_v, out_v, sem):
    wid = lax.axis_index("s") * 2 + lax.axis_index("c")
    base = wid * PER_W
    lane = lax.iota(jnp.int32, 16)
    row_base = lane >> 2
    col_base = (lane & 3) * D

    def chunk_body(t, carry):
        cb = base + t * CH
        pltpu.sync_copy(idx_hbm.at[pl.ds(cb, CH)], idx_v)
        for j in range(CH // 16):
            pos = cb + (j * 16) + lane
            fid = idx_v[pl.ds(j * 16, 16)] + (pos % F) * V
            fgrp_v[j // 8, pl.ds((j % 8) * 16, 16)] = fid >> 2
            soff_v[pl.ds(j * 16, 16)] = (fid & 3) * D
        copies = [
            pltpu.async_copy(
                table_hbm.at[fgrp_v.at[r]],
                grp_v.at[pl.ds(r * 128, 128)],
                sem,
            )
            for r in range(NSTREAM)
        ]
        for c in copies:
            c.wait()
        for jg in range(CH // 16):
            j_vec = jg * 16 + lane
            s_vec = soff_v[pl.ds(jg * 16, 16)]
            o_row = jg * 4 + row_base
            for d in range(D):
                vals = plsc.load_gather(grp_v, [j_vec, s_vec + d])
                plsc.store_scatter(out_v, [o_row, col_base + d], vals)
        ob = pl.multiple_of(cb // 4, 128)
        pltpu.sync_copy(out_v, out_hbm.at[pl.ds(ob, CH // 4)])
        return carry

    lax.fori_loop(0, NCH, chunk_body, 0)


def kernel(indices, tables):
    grp_tables = tables.reshape(G, 128)
    flat_idx = indices.reshape(BF)
    mesh = plsc.VectorSubcoreMesh(core_axis_name="c", subcore_axis_name="s")
    run = functools.partial(
        pl.kernel,
        mesh=mesh,
        out_type=jax.ShapeDtypeStruct((BF // 4, 128), jnp.float32),
        scratch_types=[
            pltpu.VMEM((CH,), jnp.int32),
            pltpu.VMEM((NSTREAM, 128), jnp.int32),
            pltpu.VMEM((CH,), jnp.int32),
            pltpu.VMEM((CH, 128), jnp.float32),
            pltpu.VMEM((CH // 4, 128), jnp.float32),
            pltpu.SemaphoreType.DMA,
        ],
        compiler_params=pltpu.CompilerParams(needs_layout_passes=False),
    )(_gather_kernel)
    out = run(grp_tables, flat_idx)
    return out.reshape(B, F * D)


# speedup vs baseline: 2.0609x; 2.0609x over previous
import functools

import jax
import jax.numpy as jnp
from jax import lax
from jax.experimental import pallas as pl
from jax.experimental.pallas import tpu as pltpu
from jax.experimental.pallas import tpu_sc as plsc

B = 16384
F = 26
V = 100000
D = 32
C = F * D
NBAND = C // 8
ROUNDS = NBAND // 2
BH = B // 2
VA = 99968
PIECES = (20096, 20096, 20096, 20096, 19584)


def _gather_kernel(table_hbm, idx_hbm, tail_hbm, out_hbm, inb, outb, row_v,
                   idxb, out_v, sem):
    cid = lax.axis_index("c")
    sid = lax.axis_index("s")

    def round_body(r, carry):
        gb = cid * ROUNDS + r

        rb = pl.multiple_of(gb * 8, 8)
        sub = sid & 7
        bhalf = sid >> 3
        c = gb * 8 + sub
        off = 0
        for pw in PIECES:
            @pl.when(sid == 0)
            def _():
                pltpu.sync_copy(table_hbm.at[pl.ds(rb, 8), pl.ds(off, pw)],
                                inb.at[:, pl.ds(0, pw)])
            plsc.subcore_barrier()
            pltpu.sync_copy(inb.at[sub, pl.ds(0, pw)],
                            row_v.at[pl.ds(off, pw)])
            plsc.subcore_barrier()
            off += pw
        tb = pl.multiple_of(c * 32, 8)
        pltpu.sync_copy(tail_hbm.at[pl.ds(tb, 32)], row_v.at[pl.ds(VA, 32)])
        f = c >> 5
        ib = pl.multiple_of(f * B + bhalf * BH, 8)
        pltpu.sync_copy(idx_hbm.at[pl.ds(ib, BH)], idxb)

        def gather_body(i, carry2):
            for k in range(16):
                ids = idxb[pl.ds(i * 256 + k * 16, 16)]
                vals = plsc.load_gather(row_v, [ids])
                out_v[pl.ds(i * 256 + k * 16, 16)] = vals
            return carry2

        lax.fori_loop(0, BH // 256, gather_body, 0)
        @pl.when(bhalf == 0)
        def _():
            pltpu.sync_copy(out_v, outb.at[sub])

        plsc.subcore_barrier()

        @pl.when(sid == 0)
        def _():
            pltpu.sync_copy(outb, out_hbm.at[pl.ds(rb, 8), pl.ds(0, BH)])

        plsc.subcore_barrier()

        @pl.when(bhalf == 1)
        def _():
            pltpu.sync_copy(out_v, outb.at[sub])

        plsc.subcore_barrier()

        @pl.when(sid == 8)
        def _():
            pltpu.sync_copy(outb, out_hbm.at[pl.ds(rb, 8), pl.ds(BH, BH)])

        return carry

    lax.fori_loop(0, ROUNDS, round_body, 0)


def kernel(indices, tables):
    rows = tables.transpose(0, 2, 1).reshape(C, V)
    tail = tables[:, VA:, :].transpose(0, 2, 1).reshape(C * (V - VA))
    idx_t = indices.T.reshape(F * B)
    mesh = plsc.VectorSubcoreMesh(core_axis_name="c", subcore_axis_name="s")
    run = functools.partial(
        pl.kernel,
        mesh=mesh,
        out_type=jax.ShapeDtypeStruct((C, B), jnp.float32),
        scratch_types=[
            pltpu.VMEM_SHARED((8, 20096), jnp.float32),
            pltpu.VMEM_SHARED((8, BH), jnp.float32),
            pltpu.VMEM((V,), jnp.float32),
            pltpu.VMEM((BH,), jnp.int32),
            pltpu.VMEM((BH,), jnp.float32),
            pltpu.SemaphoreType.DMA,
        ],
        compiler_params=pltpu.CompilerParams(needs_layout_passes=False),
    )(_gather_kernel)
    out = run(rows, idx_t, tail)
    return out.T
